# baseline (speedup 1.0000x reference)
import functools

import jax
import jax.numpy as jnp
import numpy as np
from jax import lax
from jax.experimental import pallas as pl
from jax.experimental.pallas import tpu as pltpu
from jax.experimental.pallas import tpu_sc as plsc

NC = 2
NS = 16
NW = NC * NS
LANE = 16

_SC_PARAMS = pltpu.CompilerParams(use_tc_tiling_on_sc=False)
_MESH = dict(core_axis_name="c", subcore_axis_name="s",
             num_cores=NC, num_subcores=NS)

_MASK_HI = np.uint32(0xFFFF0000)


def _wid():
    return lax.axis_index("s") * NC + lax.axis_index("c")


def _pack_body(vchunk, nchunks, table_hbm, out_hbm,
               in0, in1, o0, o1, isem0, isem1, osem0, osem1):
    embed = table_hbm.shape[1]
    half = embed // 2
    nout = half // LANE
    ins, outs = (in0, in1), (o0, o1)
    isems, osems = (isem0, isem1), (osem0, osem1)
    base = _wid() * vchunk * nchunks

    def fire(k, p):
        pltpu.async_copy(table_hbm.at[pl.ds(base + k * vchunk, vchunk)],
                         ins[p], isems[p])

    def drain_in(p):
        pltpu.make_async_copy(table_hbm.at[pl.ds(base, vchunk)],
                              ins[p], isems[p]).wait()

    def put(k, p):
        pltpu.async_copy(outs[p], out_hbm.at[pl.ds(base + k * vchunk, vchunk)],
                         osems[p])

    def drain_out(p):
        pltpu.make_async_copy(outs[p], out_hbm.at[pl.ds(base, vchunk)],
                              osems[p]).wait()

    fire(0, 0)
    fire(1, 1)

    def compute(p):
        @plsc.parallel_loop(0, vchunk, unroll=4)
        def _(r):
            for g in range(nout):
                lo = lax.bitcast_convert_type(
                    ins[p][r, pl.ds(g * LANE, LANE)], jnp.uint32)
                hi = lax.bitcast_convert_type(
                    ins[p][r, pl.ds(half + g * LANE, LANE)], jnp.uint32)
                outs[p][r, pl.ds(g * LANE, LANE)] = (
                    (lo >> 16) | (hi & _MASK_HI))

    def chunk(k, _):
        for p in range(2):
            kk = 2 * k + p
            drain_in(p)

            @pl.when(kk >= 2)
            def _(p=p):
                drain_out(p)

            compute(p)
            put(kk, p)

            nk = kk + 2

            @pl.when(nk < nchunks)
            def _(nk=nk, p=p):
                fire(nk, p)
        return 0

    lax.fori_loop(0, nchunks // 2, chunk, 0)
    if nchunks % 2:
        drain_in(0)
        drain_out(0)
        compute(0)
        put(nchunks - 1, 0)
    drain_out(0)
    drain_out(1)


def _sc_pack(table):
    vocab, embed = table.shape
    per_w = vocab // NW
    vchunk, nchunks = 125, 25
    assert per_w == vchunk * nchunks and nchunks % 2 == 1
    mesh = plsc.VectorSubcoreMesh(**_MESH)
    return functools.partial(
        pl.kernel,
        out_type=jax.ShapeDtypeStruct((vocab, embed // 2), jnp.uint32),
        mesh=mesh,
        scratch_types=[
            pltpu.VMEM((vchunk, embed), jnp.float32),
            pltpu.VMEM((vchunk, embed), jnp.float32),
            pltpu.VMEM((vchunk, embed // 2), jnp.uint32),
            pltpu.VMEM((vchunk, embed // 2), jnp.uint32),
            pltpu.SemaphoreType.DMA,
            pltpu.SemaphoreType.DMA,
            pltpu.SemaphoreType.DMA,
            pltpu.SemaphoreType.DMA,
        ],
        compiler_params=_SC_PARAMS,
    )(functools.partial(_pack_body, vchunk, nchunks))(table)


def _seq_chunks(seq):
    chunks = [128] * (seq // 128)
    if seq % 128:
        chunks.append(seq % 128)
    return chunks


ACC_UNROLL = 8


def _pool_body(seq, embed, bpw, ids_hbm, table_hbm, out_hbm,
               idx_v, buf0, buf1, out_v, sem0, sem1):
    nvec = embed // LANE
    nhalf = embed // (2 * LANE)
    chunks = _seq_chunks(seq)
    bufs, sems = (buf0, buf1), (sem0, sem1)
    base = _wid() * bpw
    pltpu.sync_copy(ids_hbm.at[pl.ds(base, bpw)], idx_v)

    def fire(b, p):
        dst = 0
        for c in chunks:
            pltpu.async_copy(table_hbm.at[idx_v.at[b, pl.ds(dst, c)]],
                             bufs[p].at[pl.ds(dst, c)], sems[p])
            dst += c

    def drain(b, p):
        dst = 0
        for c in chunks:
            pltpu.make_async_copy(table_hbm.at[idx_v.at[b, pl.ds(dst, c)]],
                                  bufs[p].at[pl.ds(dst, c)], sems[p]).wait()
            dst += c

    fire(0, 0)
    fire(1, 1)

    zeros = tuple(jnp.zeros((LANE,), jnp.float32) for _ in range(nvec))

    def acc_row(s, acc, p):
        new = list(acc)
        for c in range(nhalf):
            w = bufs[p][s, pl.ds(c * LANE, LANE)]
            lo = lax.bitcast_convert_type(w << 16, jnp.float32)
            hi = lax.bitcast_convert_type(w & _MASK_HI, jnp.float32)
            new[2 * c] = new[2 * c] + lo
            new[2 * c + 1] = new[2 * c + 1] + hi
        return tuple(new)

    def body(i, _):
        for p in range(2):
            b = 2 * i + p
            drain(b, p)
            acc = plsc.parallel_loop(0, seq, unroll=8, carry=zeros)(
                lambda s, acc, p=p: acc_row(s, acc, p))
            for j in range(nvec):
                out_v[b, pl.ds(j * LANE, LANE)] = acc[j]

            nb = b + 2

            @pl.when(nb < bpw)
            def _(nb=nb, p=p):
                fire(nb, p)
        return 0

    lax.fori_loop(0, bpw // 2, body, 0)
    pltpu.sync_copy(out_v, out_hbm.at[pl.ds(base, bpw)])


def _sc_pool(batch, seq, embed, ids, table_u32):
    bpw = batch // NW
    mesh = plsc.VectorSubcoreMesh(**_MESH)
    return functools.partial(
        pl.kernel,
        out_type=jax.ShapeDtypeStruct((batch, embed), jnp.float32),
        mesh=mesh,
        scratch_types=[
            pltpu.VMEM((bpw, seq), jnp.int32),
            pltpu.VMEM((seq, embed // 2), jnp.uint32),
            pltpu.VMEM((seq, embed // 2), jnp.uint32),
            pltpu.VMEM((bpw, embed), jnp.float32),
            pltpu.SemaphoreType.DMA,
            pltpu.SemaphoreType.DMA,
        ],
        compiler_params=_SC_PARAMS,
    )(functools.partial(_pool_body, seq, embed, bpw))(ids, table_u32)


def _mlp_body(x_ref, w1_ref, b1_ref, w2t_ref, b2_ref, o_ref):
    x = x_ref[...]
    h = jnp.dot(x, w1_ref[...], preferred_element_type=jnp.float32)
    h = jnp.maximum(h + b1_ref[...], 0.0)
    logit = jnp.sum(h * w2t_ref[...], axis=1, keepdims=True) + b2_ref[0, 0]
    o_ref[...] = 1.0 / (1.0 + jnp.exp(-logit))


def _mlp(encoded, W1, b1, W2, b2, interpret=False):
    batch, embed = encoded.shape
    hidden = W1.shape[1]
    mb = min(512, batch)
    return pl.pallas_call(
        _mlp_body,
        grid=(batch // mb,),
        in_specs=[
            pl.BlockSpec((mb, embed), lambda i: (i, 0)),
            pl.BlockSpec((embed, hidden), lambda i: (0, 0)),
            pl.BlockSpec((1, hidden), lambda i: (0, 0)),
            pl.BlockSpec((1, hidden), lambda i: (0, 0)),
            pl.BlockSpec((1, 1), lambda i: (0, 0)),
        ],
        out_specs=pl.BlockSpec((mb, 1), lambda i: (i, 0)),
        out_shape=jax.ShapeDtypeStruct((batch, 1), jnp.float32),
        interpret=interpret,
    )(encoded, W1, b1[None, :], W2.T, b2[None, :])


def _halfpack_perm(embed):
    half = embed // 2
    perm = []
    for c in range(embed // 32):
        perm.extend(range(16 * c, 16 * c + 16))
        perm.extend(range(half + 16 * c, half + 16 * c + 16))
    return perm


def kernel(text_ids, table, W1, b1, W2, b2):
    batch, seq = text_ids.shape
    vocab, embed = table.shape
    table_u32 = _sc_pack(table)
    encoded = _sc_pool(batch, seq, embed, text_ids, table_u32)
    W1p = W1[jnp.asarray(_halfpack_perm(embed)), :]
    return _mlp(encoded, W1p, b1, W2, b2)


# speedup vs baseline: 1.0194x; 1.0194x over previous
import functools

import jax
import jax.numpy as jnp
import numpy as np
from jax import lax
from jax.experimental import pallas as pl
from jax.experimental.pallas import tpu as pltpu
from jax.experimental.pallas import tpu_sc as plsc

NC = 2
NS = 16
NW = NC * NS
LANE = 16

_SC_PARAMS = pltpu.CompilerParams(use_tc_tiling_on_sc=False)
_MESH = dict(core_axis_name="c", subcore_axis_name="s",
             num_cores=NC, num_subcores=NS)

_MASK_HI = np.uint32(0xFFFF0000)


def _wid():
    return lax.axis_index("s") * NC + lax.axis_index("c")


def _pack_body(vchunk, nchunks, table_hbm, out_hbm,
               in0, in1, o0, o1, isem0, isem1, osem0, osem1):
    embed = table_hbm.shape[1]
    half = embed // 2
    nout = half // LANE
    ins, outs = (in0, in1), (o0, o1)
    isems, osems = (isem0, isem1), (osem0, osem1)
    base = _wid() * vchunk * nchunks

    def fire(k, p):
        pltpu.async_copy(table_hbm.at[pl.ds(base + k * vchunk, vchunk)],
                         ins[p], isems[p])

    def drain_in(p):
        pltpu.make_async_copy(table_hbm.at[pl.ds(base, vchunk)],
                              ins[p], isems[p]).wait()

    def put(k, p):
        pltpu.async_copy(outs[p], out_hbm.at[pl.ds(base + k * vchunk, vchunk)],
                         osems[p])

    def drain_out(p):
        pltpu.make_async_copy(outs[p], out_hbm.at[pl.ds(base, vchunk)],
                              osems[p]).wait()

    fire(0, 0)
    fire(1, 1)

    def compute(p):
        @plsc.parallel_loop(0, vchunk, unroll=4)
        def _(r):
            for g in range(nout):
                lo = lax.bitcast_convert_type(
                    ins[p][r, pl.ds(g * LANE, LANE)], jnp.uint32)
                hi = lax.bitcast_convert_type(
                    ins[p][r, pl.ds(half + g * LANE, LANE)], jnp.uint32)
                outs[p][r, pl.ds(g * LANE, LANE)] = (
                    (lo >> 16) | (hi & _MASK_HI))

    def chunk(k, _):
        for p in range(2):
            kk = 2 * k + p
            drain_in(p)

            @pl.when(kk >= 2)
            def _(p=p):
                drain_out(p)

            compute(p)
            put(kk, p)

            nk = kk + 2

            @pl.when(nk < nchunks)
            def _(nk=nk, p=p):
                fire(nk, p)
        return 0

    lax.fori_loop(0, nchunks // 2, chunk, 0)
    if nchunks % 2:
        drain_in(0)
        drain_out(0)
        compute(0)
        put(nchunks - 1, 0)
    drain_out(0)
    drain_out(1)


def _sc_pack(table):
    vocab, embed = table.shape
    per_w = vocab // NW
    vchunk, nchunks = 125, 25
    assert per_w == vchunk * nchunks and nchunks % 2 == 1
    mesh = plsc.VectorSubcoreMesh(**_MESH)
    return functools.partial(
        pl.kernel,
        out_type=jax.ShapeDtypeStruct((vocab, embed // 2), jnp.uint32),
        mesh=mesh,
        scratch_types=[
            pltpu.VMEM((vchunk, embed), jnp.float32),
            pltpu.VMEM((vchunk, embed), jnp.float32),
            pltpu.VMEM((vchunk, embed // 2), jnp.uint32),
            pltpu.VMEM((vchunk, embed // 2), jnp.uint32),
            pltpu.SemaphoreType.DMA,
            pltpu.SemaphoreType.DMA,
            pltpu.SemaphoreType.DMA,
            pltpu.SemaphoreType.DMA,
        ],
        compiler_params=_SC_PARAMS,
    )(functools.partial(_pack_body, vchunk, nchunks))(table)


def _seq_chunks(seq):
    chunks = [128] * (seq // 128)
    if seq % 128:
        chunks.append(seq % 128)
    return chunks


ACC_UNROLL = 8


def _pool_body(seq, embed, bpw, ids_hbm, table_hbm, out_hbm,
               idx_v, buf0, buf1, out_v, sem0, sem1):
    nvec = embed // LANE
    nhalf = embed // (2 * LANE)
    chunks = _seq_chunks(seq)
    bufs, sems = (buf0, buf1), (sem0, sem1)
    base = _wid() * bpw
    pltpu.sync_copy(ids_hbm.at[pl.ds(base, bpw)], idx_v)

    def fire(b, p):
        dst = 0
        for c in chunks:
            pltpu.async_copy(table_hbm.at[idx_v.at[b, pl.ds(dst, c)]],
                             bufs[p].at[pl.ds(dst, c)], sems[p])
            dst += c

    def drain(b, p):
        dst = 0
        for c in chunks:
            pltpu.make_async_copy(table_hbm.at[idx_v.at[b, pl.ds(dst, c)]],
                                  bufs[p].at[pl.ds(dst, c)], sems[p]).wait()
            dst += c

    fire(0, 0)
    fire(1, 1)

    zeros = tuple(jnp.zeros((LANE,), jnp.float32) for _ in range(nvec))

    def acc_row(s, acc, p):
        new = list(acc)
        for c in range(nhalf):
            w = bufs[p][s, pl.ds(c * LANE, LANE)]
            lo = lax.bitcast_convert_type(w << 16, jnp.float32)
            hi = lax.bitcast_convert_type(w & _MASK_HI, jnp.float32)
            new[2 * c] = new[2 * c] + lo
            new[2 * c + 1] = new[2 * c + 1] + hi
        return tuple(new)

    def body(i, _):
        for p in range(2):
            b = 2 * i + p
            drain(b, p)

            def acc_body(g, acc, p=p):
                for u in range(ACC_UNROLL):
                    acc = acc_row(g * ACC_UNROLL + u, acc, p)
                return acc

            acc = lax.fori_loop(0, seq // ACC_UNROLL, acc_body, zeros)
            for s in range(seq - seq % ACC_UNROLL, seq):
                acc = acc_row(s, acc, p)
            for j in range(nvec):
                out_v[b, pl.ds(j * LANE, LANE)] = acc[j]

            nb = b + 2

            @pl.when(nb < bpw)
            def _(nb=nb, p=p):
                fire(nb, p)
        return 0

    lax.fori_loop(0, bpw // 2, body, 0)
    pltpu.sync_copy(out_v, out_hbm.at[pl.ds(base, bpw)])


def _sc_pool(batch, seq, embed, ids, table_u32):
    bpw = batch // NW
    mesh = plsc.VectorSubcoreMesh(**_MESH)
    return functools.partial(
        pl.kernel,
        out_type=jax.ShapeDtypeStruct((batch, embed), jnp.float32),
        mesh=mesh,
        scratch_types=[
            pltpu.VMEM((bpw, seq), jnp.int32),
            pltpu.VMEM((seq, embed // 2), jnp.uint32),
            pltpu.VMEM((seq, embed // 2), jnp.uint32),
            pltpu.VMEM((bpw, embed), jnp.float32),
            pltpu.SemaphoreType.DMA,
            pltpu.SemaphoreType.DMA,
        ],
        compiler_params=_SC_PARAMS,
    )(functools.partial(_pool_body, seq, embed, bpw))(ids, table_u32)


def _mlp_body(x_ref, w1_ref, b1_ref, w2t_ref, b2_ref, o_ref):
    x = x_ref[...]
    h = jnp.dot(x, w1_ref[...], preferred_element_type=jnp.float32)
    h = jnp.maximum(h + b1_ref[...], 0.0)
    logit = jnp.sum(h * w2t_ref[...], axis=1, keepdims=True) + b2_ref[0, 0]
    o_ref[...] = 1.0 / (1.0 + jnp.exp(-logit))


def _mlp(encoded, W1, b1, W2, b2, interpret=False):
    batch, embed = encoded.shape
    hidden = W1.shape[1]
    mb = min(512, batch)
    return pl.pallas_call(
        _mlp_body,
        grid=(batch // mb,),
        in_specs=[
            pl.BlockSpec((mb, embed), lambda i: (i, 0)),
            pl.BlockSpec((embed, hidden), lambda i: (0, 0)),
            pl.BlockSpec((1, hidden), lambda i: (0, 0)),
            pl.BlockSpec((1, hidden), lambda i: (0, 0)),
            pl.BlockSpec((1, 1), lambda i: (0, 0)),
        ],
        out_specs=pl.BlockSpec((mb, 1), lambda i: (i, 0)),
        out_shape=jax.ShapeDtypeStruct((batch, 1), jnp.float32),
        interpret=interpret,
    )(encoded, W1, b1[None, :], W2.T, b2[None, :])


def _halfpack_perm(embed):
    half = embed // 2
    perm = []
    for c in range(embed // 32):
        perm.extend(range(16 * c, 16 * c + 16))
        perm.extend(range(half + 16 * c, half + 16 * c + 16))
    return perm


def kernel(text_ids, table, W1, b1, W2, b2):
    batch, seq = text_ids.shape
    vocab, embed = table.shape
    table_u32 = _sc_pack(table)
    encoded = _sc_pool(batch, seq, embed, text_ids, table_u32)
    W1p = W1[jnp.asarray(_halfpack_perm(embed)), :]
    return _mlp(encoded, W1p, b1, W2, b2)


# speedup vs baseline: 1.2408x; 1.2172x over previous
import functools

import jax
import jax.numpy as jnp
import numpy as np
from jax import lax
from jax.experimental import pallas as pl
from jax.experimental.pallas import tpu as pltpu
from jax.experimental.pallas import tpu_sc as plsc

NC = 2
NS = 16
NW = NC * NS
LANE = 16

_SC_PARAMS = pltpu.CompilerParams(use_tc_tiling_on_sc=False)
_MESH = dict(core_axis_name="c", subcore_axis_name="s",
             num_cores=NC, num_subcores=NS)

_MASK_HI = np.uint32(0xFFFF0000)


def _wid():
    return lax.axis_index("s") * NC + lax.axis_index("c")


def _pack_body(vchunk, nchunks, table_hbm, out_hbm,
               in0, in1, o0, o1, isem0, isem1, osem0, osem1):
    embed = table_hbm.shape[1]
    half = embed // 2
    nout = half // LANE
    ins, outs = (in0, in1), (o0, o1)
    isems, osems = (isem0, isem1), (osem0, osem1)
    base = _wid() * vchunk * nchunks

    def fire(k, p):
        pltpu.async_copy(table_hbm.at[pl.ds(base + k * vchunk, vchunk)],
                         ins[p], isems[p])

    def drain_in(p):
        pltpu.make_async_copy(table_hbm.at[pl.ds(base, vchunk)],
                              ins[p], isems[p]).wait()

    def put(k, p):
        pltpu.async_copy(outs[p], out_hbm.at[pl.ds(base + k * vchunk, vchunk)],
                         osems[p])

    def drain_out(p):
        pltpu.make_async_copy(outs[p], out_hbm.at[pl.ds(base, vchunk)],
                              osems[p]).wait()

    fire(0, 0)
    fire(1, 1)

    def compute(p):
        @plsc.parallel_loop(0, vchunk, unroll=4)
        def _(r):
            for g in range(nout):
                lo = lax.bitcast_convert_type(
                    ins[p][r, pl.ds(g * LANE, LANE)], jnp.uint32)
                hi = lax.bitcast_convert_type(
                    ins[p][r, pl.ds(half + g * LANE, LANE)], jnp.uint32)
                outs[p][r, pl.ds(g * LANE, LANE)] = (
                    (lo >> 16) | (hi & _MASK_HI))

    def chunk(k, _):
        for p in range(2):
            kk = 2 * k + p
            drain_in(p)

            @pl.when(kk >= 2)
            def _(p=p):
                drain_out(p)

            compute(p)
            put(kk, p)

            nk = kk + 2

            @pl.when(nk < nchunks)
            def _(nk=nk, p=p):
                fire(nk, p)
        return 0

    lax.fori_loop(0, nchunks // 2, chunk, 0)
    if nchunks % 2:
        drain_in(0)
        drain_out(0)
        compute(0)
        put(nchunks - 1, 0)
    drain_out(0)
    drain_out(1)


def _sc_pack(table):
    vocab, embed = table.shape
    per_w = vocab // NW
    vchunk, nchunks = 125, 25
    assert per_w == vchunk * nchunks and nchunks % 2 == 1
    mesh = plsc.VectorSubcoreMesh(**_MESH)
    return functools.partial(
        pl.kernel,
        out_type=jax.ShapeDtypeStruct((vocab, embed // 2), jnp.uint32),
        mesh=mesh,
        scratch_types=[
            pltpu.VMEM((vchunk, embed), jnp.float32),
            pltpu.VMEM((vchunk, embed), jnp.float32),
            pltpu.VMEM((vchunk, embed // 2), jnp.uint32),
            pltpu.VMEM((vchunk, embed // 2), jnp.uint32),
            pltpu.SemaphoreType.DMA,
            pltpu.SemaphoreType.DMA,
            pltpu.SemaphoreType.DMA,
            pltpu.SemaphoreType.DMA,
        ],
        compiler_params=_SC_PARAMS,
    )(functools.partial(_pack_body, vchunk, nchunks))(table)


def _seq_chunks(seq):
    chunks = [128] * (seq // 128)
    if seq % 128:
        chunks.append(seq % 128)
    return chunks


ACC_UNROLL = 8


NBUF = 4


def _pool_body(seq, embed, bpw, ids_hbm, table_hbm, out_hbm,
               idx_v, buf0, buf1, buf2, buf3, out_v,
               sem0, sem1, sem2, sem3):
    nvec = embed // LANE
    nhalf = embed // (2 * LANE)
    chunks = _seq_chunks(seq)
    bufs, sems = (buf0, buf1, buf2, buf3), (sem0, sem1, sem2, sem3)
    base = _wid() * bpw
    pltpu.sync_copy(ids_hbm.at[pl.ds(base, bpw)], idx_v)

    def fire(b, p):
        dst = 0
        for c in chunks:
            pltpu.async_copy(table_hbm.at[idx_v.at[b, pl.ds(dst, c)]],
                             bufs[p].at[pl.ds(dst, c)], sems[p])
            dst += c

    def drain(b, p):
        dst = 0
        for c in chunks:
            pltpu.make_async_copy(table_hbm.at[idx_v.at[b, pl.ds(dst, c)]],
                                  bufs[p].at[pl.ds(dst, c)], sems[p]).wait()
            dst += c

    for p in range(NBUF):
        fire(p, p)

    zeros = tuple(jnp.zeros((LANE,), jnp.float32) for _ in range(nvec))

    def acc_row(s, acc, p):
        new = list(acc)
        for c in range(nhalf):
            w = bufs[p][s, pl.ds(c * LANE, LANE)]
            lo = lax.bitcast_convert_type(w << 16, jnp.float32)
            hi = lax.bitcast_convert_type(w & _MASK_HI, jnp.float32)
            new[2 * c] = new[2 * c] + lo
            new[2 * c + 1] = new[2 * c + 1] + hi
        return tuple(new)

    def body(i, _):
        for p in range(NBUF):
            b = NBUF * i + p
            drain(b, p)

            def acc_body(g, acc, p=p):
                for u in range(ACC_UNROLL):
                    acc = acc_row(g * ACC_UNROLL + u, acc, p)
                return acc

            acc = lax.fori_loop(0, seq // ACC_UNROLL, acc_body, zeros)
            for s in range(seq - seq % ACC_UNROLL, seq):
                acc = acc_row(s, acc, p)
            for j in range(nvec):
                out_v[b, pl.ds(j * LANE, LANE)] = acc[j]

            nb = b + NBUF

            @pl.when(nb < bpw)
            def _(nb=nb, p=p):
                fire(nb, p)
        return 0

    lax.fori_loop(0, bpw // NBUF, body, 0)
    pltpu.sync_copy(out_v, out_hbm.at[pl.ds(base, bpw)])


def _sc_pool(batch, seq, embed, ids, table_u32):
    bpw = batch // NW
    mesh = plsc.VectorSubcoreMesh(**_MESH)
    return functools.partial(
        pl.kernel,
        out_type=jax.ShapeDtypeStruct((batch, embed), jnp.float32),
        mesh=mesh,
        scratch_types=(
            [pltpu.VMEM((bpw, seq), jnp.int32)]
            + [pltpu.VMEM((seq, embed // 2), jnp.uint32)] * NBUF
            + [pltpu.VMEM((bpw, embed), jnp.float32)]
            + [pltpu.SemaphoreType.DMA] * NBUF
        ),
        compiler_params=_SC_PARAMS,
    )(functools.partial(_pool_body, seq, embed, bpw))(ids, table_u32)


def _mlp_body(x_ref, w1_ref, b1_ref, w2t_ref, b2_ref, o_ref):
    x = x_ref[...]
    h = jnp.dot(x, w1_ref[...], preferred_element_type=jnp.float32)
    h = jnp.maximum(h + b1_ref[...], 0.0)
    logit = jnp.sum(h * w2t_ref[...], axis=1, keepdims=True) + b2_ref[0, 0]
    o_ref[...] = 1.0 / (1.0 + jnp.exp(-logit))


def _mlp(encoded, W1, b1, W2, b2, interpret=False):
    batch, embed = encoded.shape
    hidden = W1.shape[1]
    mb = min(512, batch)
    return pl.pallas_call(
        _mlp_body,
        grid=(batch // mb,),
        in_specs=[
            pl.BlockSpec((mb, embed), lambda i: (i, 0)),
            pl.BlockSpec((embed, hidden), lambda i: (0, 0)),
            pl.BlockSpec((1, hidden), lambda i: (0, 0)),
            pl.BlockSpec((1, hidden), lambda i: (0, 0)),
            pl.BlockSpec((1, 1), lambda i: (0, 0)),
        ],
        out_specs=pl.BlockSpec((mb, 1), lambda i: (i, 0)),
        out_shape=jax.ShapeDtypeStruct((batch, 1), jnp.float32),
        interpret=interpret,
    )(encoded, W1, b1[None, :], W2.T, b2[None, :])


def _halfpack_perm(embed):
    half = embed // 2
    perm = []
    for c in range(embed // 32):
        perm.extend(range(16 * c, 16 * c + 16))
        perm.extend(range(half + 16 * c, half + 16 * c + 16))
    return perm


def kernel(text_ids, table, W1, b1, W2, b2):
    batch, seq = text_ids.shape
    vocab, embed = table.shape
    table_u32 = _sc_pack(table)
    encoded = _sc_pool(batch, seq, embed, text_ids, table_u32)
    W1p = W1[jnp.asarray(_halfpack_perm(embed)), :]
    return _mlp(encoded, W1p, b1, W2, b2)


# speedup vs baseline: 1.2680x; 1.0220x over previous
import functools

import jax
import jax.numpy as jnp
import numpy as np
from jax import lax
from jax.experimental import pallas as pl
from jax.experimental.pallas import tpu as pltpu
from jax.experimental.pallas import tpu_sc as plsc

NC = 2
NS = 16
NW = NC * NS
LANE = 16

_SC_PARAMS = pltpu.CompilerParams(use_tc_tiling_on_sc=False)
_MESH = dict(core_axis_name="c", subcore_axis_name="s",
             num_cores=NC, num_subcores=NS)

_MASK_HI = np.uint32(0xFFFF0000)


def _wid():
    return lax.axis_index("s") * NC + lax.axis_index("c")


NPBUF = 4


def _pack_body(vchunk, nchunks, table_hbm, out_hbm,
               in0, in1, in2, in3, o0, o1, o2, o3,
               isem0, isem1, isem2, isem3, osem0, osem1, osem2, osem3):
    embed = table_hbm.shape[1]
    half = embed // 2
    nout = half // LANE
    ins, outs = (in0, in1, in2, in3), (o0, o1, o2, o3)
    isems = (isem0, isem1, isem2, isem3)
    osems = (osem0, osem1, osem2, osem3)
    base = _wid() * vchunk * nchunks

    def fire(k, p):
        pltpu.async_copy(table_hbm.at[pl.ds(base + k * vchunk, vchunk)],
                         ins[p], isems[p])

    def drain_in(p):
        pltpu.make_async_copy(table_hbm.at[pl.ds(base, vchunk)],
                              ins[p], isems[p]).wait()

    def put(k, p):
        pltpu.async_copy(outs[p], out_hbm.at[pl.ds(base + k * vchunk, vchunk)],
                         osems[p])

    def drain_out(p):
        pltpu.make_async_copy(outs[p], out_hbm.at[pl.ds(base, vchunk)],
                              osems[p]).wait()

    for p in range(NPBUF):
        fire(p, p)

    def compute(p):
        @plsc.parallel_loop(0, vchunk, unroll=4)
        def _(r):
            for g in range(nout):
                lo = lax.bitcast_convert_type(
                    ins[p][r, pl.ds(g * LANE, LANE)], jnp.uint32)
                hi = lax.bitcast_convert_type(
                    ins[p][r, pl.ds(half + g * LANE, LANE)], jnp.uint32)
                outs[p][r, pl.ds(g * LANE, LANE)] = (
                    (lo >> 16) | (hi & _MASK_HI))

    def chunk(k, _):
        for p in range(NPBUF):
            kk = NPBUF * k + p
            drain_in(p)

            @pl.when(kk >= NPBUF)
            def _(p=p):
                drain_out(p)

            compute(p)
            put(kk, p)

            nk = kk + NPBUF

            @pl.when(nk < nchunks)
            def _(nk=nk, p=p):
                fire(nk, p)
        return 0

    lax.fori_loop(0, nchunks // NPBUF, chunk, 0)
    for t in range(nchunks - nchunks % NPBUF, nchunks):
        p = t % NPBUF
        drain_in(p)
        drain_out(p)
        compute(p)
        put(t, p)
    for p in range(NPBUF):
        drain_out(p)


def _sc_pack(table):
    vocab, embed = table.shape
    per_w = vocab // NW
    vchunk, nchunks = 125, 25
    assert per_w == vchunk * nchunks
    mesh = plsc.VectorSubcoreMesh(**_MESH)
    return functools.partial(
        pl.kernel,
        out_type=jax.ShapeDtypeStruct((vocab, embed // 2), jnp.uint32),
        mesh=mesh,
        scratch_types=(
            [pltpu.VMEM((vchunk, embed), jnp.float32)] * NPBUF
            + [pltpu.VMEM((vchunk, embed // 2), jnp.uint32)] * NPBUF
            + [pltpu.SemaphoreType.DMA] * 2 * NPBUF
        ),
        compiler_params=_SC_PARAMS,
    )(functools.partial(_pack_body, vchunk, nchunks))(table)


def _seq_chunks(seq):
    chunks = [128] * (seq // 128)
    if seq % 128:
        chunks.append(seq % 128)
    return chunks


ACC_UNROLL = 8


NBUF = 4


def _pool_body(seq, embed, bpw, ids_hbm, table_hbm, out_hbm,
               idx_v, buf0, buf1, buf2, buf3, out_v,
               sem0, sem1, sem2, sem3):
    nvec = embed // LANE
    nhalf = embed // (2 * LANE)
    chunks = _seq_chunks(seq)
    bufs, sems = (buf0, buf1, buf2, buf3), (sem0, sem1, sem2, sem3)
    base = _wid() * bpw
    pltpu.sync_copy(ids_hbm.at[pl.ds(base, bpw)], idx_v)

    def fire(b, p):
        dst = 0
        for c in chunks:
            pltpu.async_copy(table_hbm.at[idx_v.at[b, pl.ds(dst, c)]],
                             bufs[p].at[pl.ds(dst, c)], sems[p])
            dst += c

    def drain(b, p):
        dst = 0
        for c in chunks:
            pltpu.make_async_copy(table_hbm.at[idx_v.at[b, pl.ds(dst, c)]],
                                  bufs[p].at[pl.ds(dst, c)], sems[p]).wait()
            dst += c

    for p in range(NBUF):
        fire(p, p)

    zeros = tuple(jnp.zeros((LANE,), jnp.float32) for _ in range(nvec))

    def acc_row(s, acc, p):
        new = list(acc)
        for c in range(nhalf):
            w = bufs[p][s, pl.ds(c * LANE, LANE)]
            lo = lax.bitcast_convert_type(w << 16, jnp.float32)
            hi = lax.bitcast_convert_type(w & _MASK_HI, jnp.float32)
            new[2 * c] = new[2 * c] + lo
            new[2 * c + 1] = new[2 * c + 1] + hi
        return tuple(new)

    def body(i, _):
        for p in range(NBUF):
            b = NBUF * i + p
            drain(b, p)

            def acc_body(g, acc, p=p):
                for u in range(ACC_UNROLL):
                    acc = acc_row(g * ACC_UNROLL + u, acc, p)
                return acc

            acc = lax.fori_loop(0, seq // ACC_UNROLL, acc_body, zeros)
            for s in range(seq - seq % ACC_UNROLL, seq):
                acc = acc_row(s, acc, p)
            for j in range(nvec):
                out_v[b, pl.ds(j * LANE, LANE)] = acc[j]

            nb = b + NBUF

            @pl.when(nb < bpw)
            def _(nb=nb, p=p):
                fire(nb, p)
        return 0

    lax.fori_loop(0, bpw // NBUF, body, 0)
    pltpu.sync_copy(out_v, out_hbm.at[pl.ds(base, bpw)])


def _sc_pool(batch, seq, embed, ids, table_u32):
    bpw = batch // NW
    mesh = plsc.VectorSubcoreMesh(**_MESH)
    return functools.partial(
        pl.kernel,
        out_type=jax.ShapeDtypeStruct((batch, embed), jnp.float32),
        mesh=mesh,
        scratch_types=(
            [pltpu.VMEM((bpw, seq), jnp.int32)]
            + [pltpu.VMEM((seq, embed // 2), jnp.uint32)] * NBUF
            + [pltpu.VMEM((bpw, embed), jnp.float32)]
            + [pltpu.SemaphoreType.DMA] * NBUF
        ),
        compiler_params=_SC_PARAMS,
    )(functools.partial(_pool_body, seq, embed, bpw))(ids, table_u32)


def _mlp_body(x_ref, w1_ref, b1_ref, w2t_ref, b2_ref, o_ref):
    x = x_ref[...]
    h = jnp.dot(x, w1_ref[...], preferred_element_type=jnp.float32)
    h = jnp.maximum(h + b1_ref[...], 0.0)
    logit = jnp.sum(h * w2t_ref[...], axis=1, keepdims=True) + b2_ref[0, 0]
    o_ref[...] = 1.0 / (1.0 + jnp.exp(-logit))


def _mlp(encoded, W1, b1, W2, b2, interpret=False):
    batch, embed = encoded.shape
    hidden = W1.shape[1]
    mb = min(512, batch)
    return pl.pallas_call(
        _mlp_body,
        grid=(batch // mb,),
        in_specs=[
            pl.BlockSpec((mb, embed), lambda i: (i, 0)),
            pl.BlockSpec((embed, hidden), lambda i: (0, 0)),
            pl.BlockSpec((1, hidden), lambda i: (0, 0)),
            pl.BlockSpec((1, hidden), lambda i: (0, 0)),
            pl.BlockSpec((1, 1), lambda i: (0, 0)),
        ],
        out_specs=pl.BlockSpec((mb, 1), lambda i: (i, 0)),
        out_shape=jax.ShapeDtypeStruct((batch, 1), jnp.float32),
        interpret=interpret,
    )(encoded, W1, b1[None, :], W2.T, b2[None, :])


def _halfpack_perm(embed):
    half = embed // 2
    perm = []
    for c in range(embed // 32):
        perm.extend(range(16 * c, 16 * c + 16))
        perm.extend(range(half + 16 * c, half + 16 * c + 16))
    return perm


def kernel(text_ids, table, W1, b1, W2, b2):
    batch, seq = text_ids.shape
    vocab, embed = table.shape
    table_u32 = _sc_pack(table)
    encoded = _sc_pool(batch, seq, embed, text_ids, table_u32)
    W1p = W1[jnp.asarray(_halfpack_perm(embed)), :]
    return _mlp(encoded, W1p, b1, W2, b2)
